# baseline (speedup 1.0000x reference)
import functools

import jax
import jax.numpy as jnp
from jax import lax
from jax.experimental import pallas as pl
from jax.experimental.pallas import tpu as pltpu
from jax.experimental.pallas import tpu_sc as plsc

N = 10000
C = 128
NC, NS = 2, 16
NW = NC * NS
CHUNK = 128
NBUF = 2
N_PAD = 10112
RPW = N_PAD // NS
DEG_W = 16
DEG_WIN = 8
ROW_TILE = 400

_mesh = plsc.VectorSubcoreMesh(
    core_axis_name="c", subcore_axis_name="s", num_cores=NC, num_subcores=NS
)


def _worker_id():
    return lax.axis_index("s") * NC + lax.axis_index("c")


def _deg_kernel(nch):
    @functools.partial(
        pl.kernel,
        out_type=jax.ShapeDtypeStruct((NC, N_PAD, DEG_W), jnp.float32),
        mesh=_mesh,
        scratch_types=[
            pltpu.VMEM((nch, CHUNK), jnp.int32),
            pltpu.VMEM((CHUNK, DEG_W), jnp.float32),
            pltpu.VMEM_SHARED((N_PAD, DEG_W), jnp.float32),
            pltpu.SemaphoreType.DMA,
        ],
    )
    def k(dst_hbm, zeros_hbm, out_hbm, dst_v, ones_v, acc_sh, sem):
        cid = lax.axis_index("c")
        sid = lax.axis_index("s")
        wid = _worker_id()

        pltpu.sync_copy(dst_hbm.at[wid], dst_v)
        pltpu.sync_copy(
            zeros_hbm.at[pl.ds(sid * RPW, RPW)], acc_sh.at[pl.ds(sid * RPW, RPW)]
        )

        @pl.loop(0, CHUNK)
        def _(r):
            ones_v[r] = jnp.full((DEG_W,), 1.0, jnp.float32)

        plsc.subcore_barrier()

        def start(c):
            pltpu.make_async_copy(ones_v, acc_sh.at[dst_v.at[c]], sem).start(
                add=True
            )

        def drain(c):
            pltpu.make_async_copy(ones_v, acc_sh.at[dst_v.at[c]], sem).wait()

        @pl.loop(0, DEG_WIN)
        def _(c):
            start(c)

        @pl.loop(0, nch - DEG_WIN)
        def _(c):
            drain(c)
            start(c + DEG_WIN)

        @pl.loop(nch - DEG_WIN, nch)
        def _(c):
            drain(c)

        plsc.subcore_barrier()
        pltpu.sync_copy(
            acc_sh.at[pl.ds(sid * RPW, RPW)],
            out_hbm.at[cid, pl.ds(sid * RPW, RPW)],
        )

    return k


def _agg_kernel(nch):
    assert nch % 4 == 0 and nch >= 8
    ih = nch // 2

    @functools.partial(
        pl.kernel,
        out_type=jax.ShapeDtypeStruct((NC, N_PAD, C), jnp.float32),
        mesh=_mesh,
        scratch_types=[
            pltpu.VMEM((ih, CHUNK), jnp.int32),
            pltpu.VMEM((ih, CHUNK), jnp.int32),
            [pltpu.VMEM((CHUNK, C), jnp.float32)] * NBUF,
            pltpu.VMEM_SHARED((N_PAD, C), jnp.float32),
            [pltpu.SemaphoreType.DMA] * NBUF,
            [pltpu.SemaphoreType.DMA] * NBUF,
        ],
    )
    def k(src_hbm, dst_hbm, table_hbm, zeros_hbm, out_hbm,
          src_v, dst_v, rows, acc_sh, sg, ss):
        cid = lax.axis_index("c")
        sid = lax.axis_index("s")
        wid = _worker_id()

        pltpu.sync_copy(
            zeros_hbm.at[pl.ds(sid * RPW, RPW)], acc_sh.at[pl.ds(sid * RPW, RPW)]
        )
        plsc.subcore_barrier()

        def g_desc(c, b):
            return pltpu.make_async_copy(table_hbm.at[src_v.at[c]], rows[b], sg[b])

        def s_desc(c, b):
            return pltpu.make_async_copy(rows[b], acc_sh.at[dst_v.at[c]], ss[b])

        for h in range(2):
            pltpu.sync_copy(src_hbm.at[wid, pl.ds(h * ih, ih)], src_v)
            pltpu.sync_copy(dst_hbm.at[wid, pl.ds(h * ih, ih)], dst_v)

            g_desc(0, 0).start()
            g_desc(0, 0).wait()
            s_desc(0, 0).start(add=True)
            g_desc(1, 1).start()

            @pl.loop(0, (ih - 2) // 2)
            def _(g):
                c0 = 1 + 2 * g
                for j in range(2):
                    c = c0 + j
                    b = (1 + j) % 2
                    g_desc(c, b).wait()
                    s_desc(c, b).start(add=True)
                    s_desc(c - 1, 1 - b).wait()
                    g_desc(c + 1, 1 - b).start()

            b_last = (ih - 1) % 2
            g_desc(ih - 1, b_last).wait()
            s_desc(ih - 1, b_last).start(add=True)
            s_desc(ih - 2, 1 - b_last).wait()
            s_desc(ih - 1, b_last).wait()

        plsc.subcore_barrier()
        pltpu.sync_copy(
            acc_sh.at[pl.ds(sid * RPW, RPW)],
            out_hbm.at[cid, pl.ds(sid * RPW, RPW)],
        )

    return k


def _s_from_deg(deg0_ref, deg1_ref):
    deg = deg0_ref[0, :, :1] + deg1_ref[0, :, :1] + 1.0
    return lax.rsqrt(deg)


def _mm_scale_body(x_ref, w_ref, deg0_ref, deg1_ref, out_ref):
    s = _s_from_deg(deg0_ref, deg1_ref)
    h = jnp.dot(
        x_ref[...], w_ref[...],
        preferred_element_type=jnp.float32, precision=lax.Precision.HIGHEST,
    )
    out_ref[...] = h * s


def _combine_mm_body(p_ref0, p_ref1, h2_ref, deg0_ref, deg1_ref, b_ref, w_ref,
                     out_ref):
    s = _s_from_deg(deg0_ref, deg1_ref)
    t = s * (p_ref0[0] + p_ref1[0] + h2_ref[...]) + b_ref[...]
    g = jnp.maximum(t, 0.0)
    h = jnp.dot(
        g, w_ref[...],
        preferred_element_type=jnp.float32, precision=lax.Precision.HIGHEST,
    )
    out_ref[...] = h * s


def _final_body(p_ref0, p_ref1, h2_ref, deg0_ref, deg1_ref, b_ref, out_ref):
    s = _s_from_deg(deg0_ref, deg1_ref)
    out_ref[...] = s * (p_ref0[0] + p_ref1[0] + h2_ref[...]) + b_ref[...]


def _row_spec(w):
    return pl.BlockSpec((ROW_TILE, w), lambda i: (i, 0))


def _part_spec(core, w):
    return pl.BlockSpec((1, ROW_TILE, w), lambda i, _c=core: (_c, i, 0))


def _full_spec(r, w):
    return pl.BlockSpec((r, w), lambda i: (0, 0))


_GRID = (N // ROW_TILE,)
_F32 = jnp.float32


@jax.jit
def kernel(x, edge_index, W1, b1, W2, b2):
    ei = edge_index.astype(jnp.int32)
    src, dst = ei[0], ei[1]
    e = src.shape[0]
    grp = NW * CHUNK * 4
    epw = max(2, -(-e // grp)) * CHUNK * 4
    e_pad = epw * NW
    nch = epw // CHUNK
    if e_pad != e:
        pad = e_pad - e
        src = jnp.concatenate([src, jnp.zeros((pad,), jnp.int32)])
        dst = jnp.concatenate([dst, jnp.full((pad,), N, jnp.int32)])
    src = src.reshape(NW, nch, CHUNK)
    dst = dst.reshape(NW, nch, CHUNK)

    zeros_deg = jnp.zeros((N_PAD, DEG_W), _F32)
    zeros_acc = jnp.zeros((N_PAD, C), _F32)
    b1r = b1.reshape(1, C)
    b2r = b2.reshape(1, C)

    degp = _deg_kernel(nch)(dst, zeros_deg)

    h2_1 = pl.pallas_call(
        _mm_scale_body,
        grid=_GRID,
        in_specs=[
            _row_spec(C), _full_spec(C, C),
            _part_spec(0, DEG_W), _part_spec(1, DEG_W),
        ],
        out_specs=_row_spec(C),
        out_shape=jax.ShapeDtypeStruct((N, C), _F32),
    )(x, W1, degp, degp)

    p = _agg_kernel(nch)(src, dst, h2_1, zeros_acc)

    h2_2 = pl.pallas_call(
        _combine_mm_body,
        grid=_GRID,
        in_specs=[
            _part_spec(0, C), _part_spec(1, C), _row_spec(C),
            _part_spec(0, DEG_W), _part_spec(1, DEG_W),
            _full_spec(1, C), _full_spec(C, C),
        ],
        out_specs=_row_spec(C),
        out_shape=jax.ShapeDtypeStruct((N, C), _F32),
    )(p, p, h2_1, degp, degp, b1r, W2)

    q = _agg_kernel(nch)(src, dst, h2_2, zeros_acc)

    out = pl.pallas_call(
        _final_body,
        grid=_GRID,
        in_specs=[
            _part_spec(0, C), _part_spec(1, C), _row_spec(C),
            _part_spec(0, DEG_W), _part_spec(1, DEG_W),
            _full_spec(1, C),
        ],
        out_specs=_row_spec(C),
        out_shape=jax.ShapeDtypeStruct((N, C), _F32),
    )(q, q, h2_2, degp, degp, b2r)

    return out


# speedup vs baseline: 1.2098x; 1.2098x over previous
import functools

import jax
import jax.numpy as jnp
from jax import lax
from jax.experimental import pallas as pl
from jax.experimental.pallas import tpu as pltpu
from jax.experimental.pallas import tpu_sc as plsc

N = 10000
C = 128
NC, NS = 2, 16
CHUNK = 128
N_PAD = 10112
RPW = N_PAD // NS
DEG_W = 16
DEG_WIN = 8
ROW_TILE = 2000

_mesh = plsc.VectorSubcoreMesh(
    core_axis_name="c", subcore_axis_name="s", num_cores=NC, num_subcores=NS
)


def _chunk_split(e):
    nct = -(-e // (CHUNK * 256)) * 256
    per16 = nct // 16
    w0 = min(max((int(per16 * 0.8) // 16) * 16, 16), per16 - 16)
    return nct, w0, per16 - w0


def _deg_kernel(w0, w1):
    @functools.partial(
        pl.kernel,
        out_type=jax.ShapeDtypeStruct((NC, N_PAD, DEG_W), jnp.float32),
        mesh=_mesh,
        scratch_types=[
            pltpu.VMEM((w0, CHUNK), jnp.int32),
            pltpu.VMEM((CHUNK, DEG_W), jnp.float32),
            pltpu.VMEM_SHARED((N_PAD, DEG_W), jnp.float32),
            pltpu.SemaphoreType.DMA,
        ],
    )
    def k(dst_hbm, zeros_hbm, out_hbm, dst_v, ones_v, acc_sh, sem):
        cid = lax.axis_index("c")
        sid = lax.axis_index("s")

        pltpu.sync_copy(
            zeros_hbm.at[pl.ds(sid * RPW, RPW)], acc_sh.at[pl.ds(sid * RPW, RPW)]
        )

        @pl.loop(0, CHUNK)
        def _(r):
            ones_v[r] = jnp.full((DEG_W,), 1.0, jnp.float32)

        def run(base, cnt):
            pltpu.sync_copy(dst_hbm.at[pl.ds(base, cnt)], dst_v.at[pl.ds(0, cnt)])

            def start(c):
                pltpu.make_async_copy(ones_v, acc_sh.at[dst_v.at[c]], sem).start(
                    add=True
                )

            def drain(c):
                pltpu.make_async_copy(ones_v, acc_sh.at[dst_v.at[c]], sem).wait()

            @pl.loop(0, DEG_WIN)
            def _(c):
                start(c)

            @pl.loop(0, cnt - DEG_WIN)
            def _(c):
                drain(c)
                start(c + DEG_WIN)

            @pl.loop(cnt - DEG_WIN, cnt)
            def _(c):
                drain(c)

        plsc.subcore_barrier()

        @pl.when(cid == 0)
        def _():
            run(sid * w0, w0)

        @pl.when(cid == 1)
        def _():
            run(NS * w0 + sid * w1, w1)

        plsc.subcore_barrier()
        pltpu.sync_copy(
            acc_sh.at[pl.ds(sid * RPW, RPW)],
            out_hbm.at[cid, pl.ds(sid * RPW, RPW)],
        )

    return k


def _agg_kernel(w0, w1):
    assert w0 % 16 == 0 and w1 % 16 == 0
    ih0 = w0 // 2

    @functools.partial(
        pl.kernel,
        out_type=jax.ShapeDtypeStruct((NC, N_PAD, C), jnp.float32),
        mesh=_mesh,
        scratch_types=[
            pltpu.VMEM((ih0, CHUNK), jnp.int32),
            pltpu.VMEM((ih0, CHUNK), jnp.int32),
            [pltpu.VMEM((CHUNK, C), jnp.float32)] * 2,
            pltpu.VMEM_SHARED((N_PAD, C), jnp.float32),
            [pltpu.SemaphoreType.DMA] * 2,
            [pltpu.SemaphoreType.DMA] * 2,
        ],
    )
    def k(src_hbm, dst_hbm, table_hbm, zeros_hbm, out_hbm,
          src_v, dst_v, rows, acc_sh, sg, ss):
        cid = lax.axis_index("c")
        sid = lax.axis_index("s")

        pltpu.sync_copy(
            zeros_hbm.at[pl.ds(sid * RPW, RPW)], acc_sh.at[pl.ds(sid * RPW, RPW)]
        )

        def g_desc(c, b):
            return pltpu.make_async_copy(table_hbm.at[src_v.at[c]], rows[b], sg[b])

        def s_desc(c, b):
            return pltpu.make_async_copy(rows[b], acc_sh.at[dst_v.at[c]], ss[b])

        def run(base, cnt):
            ih = cnt // 2
            for h in range(2):
                pltpu.sync_copy(
                    src_hbm.at[pl.ds(base + h * ih, ih)], src_v.at[pl.ds(0, ih)]
                )
                pltpu.sync_copy(
                    dst_hbm.at[pl.ds(base + h * ih, ih)], dst_v.at[pl.ds(0, ih)]
                )

                g_desc(0, 0).start()
                g_desc(0, 0).wait()
                s_desc(0, 0).start(add=True)
                g_desc(1, 1).start()

                @pl.loop(0, (ih - 2) // 2)
                def _(g):
                    c0 = 1 + 2 * g
                    for j in range(2):
                        c = c0 + j
                        b = (1 + j) % 2
                        g_desc(c, b).wait()
                        s_desc(c, b).start(add=True)
                        s_desc(c - 1, 1 - b).wait()
                        g_desc(c + 1, 1 - b).start()

                b_last = (ih - 1) % 2
                g_desc(ih - 1, b_last).wait()
                s_desc(ih - 1, b_last).start(add=True)
                s_desc(ih - 2, 1 - b_last).wait()
                s_desc(ih - 1, b_last).wait()

        plsc.subcore_barrier()

        @pl.when(cid == 0)
        def _():
            run(sid * w0, w0)

        @pl.when(cid == 1)
        def _():
            run(NS * w0 + sid * w1, w1)

        plsc.subcore_barrier()
        pltpu.sync_copy(
            acc_sh.at[pl.ds(sid * RPW, RPW)],
            out_hbm.at[cid, pl.ds(sid * RPW, RPW)],
        )

    return k


def _s_from_deg(deg0_ref, deg1_ref):
    deg = deg0_ref[0, :, :1] + deg1_ref[0, :, :1] + 1.0
    return lax.rsqrt(deg)


def _mm_scale_body(x_ref, w_ref, deg0_ref, deg1_ref, out_ref):
    s = _s_from_deg(deg0_ref, deg1_ref)
    h = jnp.dot(
        x_ref[...], w_ref[...],
        preferred_element_type=jnp.float32, precision=lax.Precision.HIGHEST,
    )
    out_ref[...] = h * s


def _combine_mm_body(p_ref0, p_ref1, h2_ref, deg0_ref, deg1_ref, b_ref, w_ref,
                     out_ref):
    s = _s_from_deg(deg0_ref, deg1_ref)
    t = s * (p_ref0[0] + p_ref1[0] + h2_ref[...]) + b_ref[...]
    g = jnp.maximum(t, 0.0)
    h = jnp.dot(
        g, w_ref[...],
        preferred_element_type=jnp.float32, precision=lax.Precision.HIGHEST,
    )
    out_ref[...] = h * s


def _final_body(p_ref0, p_ref1, h2_ref, deg0_ref, deg1_ref, b_ref, out_ref):
    s = _s_from_deg(deg0_ref, deg1_ref)
    out_ref[...] = s * (p_ref0[0] + p_ref1[0] + h2_ref[...]) + b_ref[...]


def _row_spec(w):
    return pl.BlockSpec((ROW_TILE, w), lambda i: (i, 0))


def _part_spec(core, w):
    return pl.BlockSpec((1, ROW_TILE, w), lambda i, _c=core: (_c, i, 0))


def _full_spec(r, w):
    return pl.BlockSpec((r, w), lambda i: (0, 0))


_GRID = (N // ROW_TILE,)
_F32 = jnp.float32


@jax.jit
def kernel(x, edge_index, W1, b1, W2, b2):
    ei = edge_index.astype(jnp.int32)
    src, dst = ei[0], ei[1]
    e = src.shape[0]
    nct, w0, w1 = _chunk_split(e)
    e_pad = nct * CHUNK
    if e_pad != e:
        pad = e_pad - e
        src = jnp.concatenate([src, jnp.zeros((pad,), jnp.int32)])
        dst = jnp.concatenate([dst, jnp.full((pad,), N, jnp.int32)])
    src = src.reshape(nct, CHUNK)
    dst = dst.reshape(nct, CHUNK)

    zeros_deg = jnp.zeros((N_PAD, DEG_W), _F32)
    zeros_acc = jnp.zeros((N_PAD, C), _F32)
    b1r = b1.reshape(1, C)
    b2r = b2.reshape(1, C)

    degp = _deg_kernel(w0, w1)(dst, zeros_deg)

    h2_1 = pl.pallas_call(
        _mm_scale_body,
        grid=_GRID,
        in_specs=[
            _row_spec(C), _full_spec(C, C),
            _part_spec(0, DEG_W), _part_spec(1, DEG_W),
        ],
        out_specs=_row_spec(C),
        out_shape=jax.ShapeDtypeStruct((N, C), _F32),
    )(x, W1, degp, degp)

    p = _agg_kernel(w0, w1)(src, dst, h2_1, zeros_acc)

    h2_2 = pl.pallas_call(
        _combine_mm_body,
        grid=_GRID,
        in_specs=[
            _part_spec(0, C), _part_spec(1, C), _row_spec(C),
            _part_spec(0, DEG_W), _part_spec(1, DEG_W),
            _full_spec(1, C), _full_spec(C, C),
        ],
        out_specs=_row_spec(C),
        out_shape=jax.ShapeDtypeStruct((N, C), _F32),
    )(p, p, h2_1, degp, degp, b1r, W2)

    q = _agg_kernel(w0, w1)(src, dst, h2_2, zeros_acc)

    out = pl.pallas_call(
        _final_body,
        grid=_GRID,
        in_specs=[
            _part_spec(0, C), _part_spec(1, C), _row_spec(C),
            _part_spec(0, DEG_W), _part_spec(1, DEG_W),
            _full_spec(1, C),
        ],
        out_specs=_row_spec(C),
        out_shape=jax.ShapeDtypeStruct((N, C), _F32),
    )(q, q, h2_2, degp, degp, b2r)

    return out
